# baseline (speedup 1.0000x reference)
import functools

import jax
import jax.numpy as jnp
from jax.experimental import pallas as pl
from jax.experimental.pallas import tpu as pltpu

N = 10000
E = 160000
EPS = 1e-5

_RB = 400


def _mm_body(x_ref, w_ref, o_ref):
    o_ref[...] = jnp.dot(x_ref[...], w_ref[...],
                         preferred_element_type=jnp.float32)


def _mm(x, w):
    n, k = x.shape
    m = w.shape[1]
    grid = (n // _RB,)
    return pl.pallas_call(
        _mm_body,
        grid=grid,
        in_specs=[
            pl.BlockSpec((_RB, k), lambda i: (i, 0)),
            pl.BlockSpec((k, m), lambda i: (0, 0)),
        ],
        out_specs=pl.BlockSpec((_RB, m), lambda i: (i, 0)),
        out_shape=jax.ShapeDtypeStruct((n, m), jnp.float32),
    )(x, w)


def _gat_layer(feat, src, dst, deg_o_isqrt, deg_i_sqrt, W, al, ar, rW, H, out):
    Nn = feat.shape[0]
    feat = feat * deg_o_isqrt[:, None]
    ft2 = _mm(feat, W)
    ft = ft2.reshape(Nn, H, out)
    el = jnp.sum(ft * al[None], axis=-1)
    er = jnp.sum(ft * ar[None], axis=-1)
    e = jax.nn.leaky_relu(el[src] + er[dst], negative_slope=0.2)
    emax = jax.ops.segment_max(e, dst, num_segments=Nn)
    emax = jnp.where(jnp.isfinite(emax), emax, 0.0)
    ee = jnp.exp(e - emax[dst])
    den = jax.ops.segment_sum(ee, dst, num_segments=Nn)
    alpha = ee / den[dst]
    rst = jax.ops.segment_sum(alpha[:, :, None] * ft[src], dst,
                              num_segments=Nn)
    rst = rst * deg_i_sqrt[:, None, None]
    res = _mm(feat, rW).reshape(Nn, H, out)
    return jax.nn.relu(rst + res)


def _bn_relu(h, g, b):
    m = jnp.mean(h, axis=0)
    v = jnp.mean((h - m) ** 2, axis=0)
    return jax.nn.relu((h - m) / jnp.sqrt(v + EPS) * g + b)


def kernel(x, edge_index, W0, al0, ar0, rW0, g0, b0, W1, al1, ar1, rW1,
           g1, b1, W2, al2, ar2, rW2, bias_last):
    src = edge_index[0]
    dst = edge_index[1]
    ones = jnp.ones((E,), dtype=jnp.float32)
    deg_o = jnp.maximum(jax.ops.segment_sum(ones, src, num_segments=N), 1.0)
    deg_i = jnp.maximum(jax.ops.segment_sum(ones, dst, num_segments=N), 1.0)
    deg_o_isqrt = deg_o ** -0.5
    deg_i_sqrt = deg_i ** 0.5

    h = _gat_layer(x, src, dst, deg_o_isqrt, deg_i_sqrt, W0, al0, ar0, rW0,
                   4, 256).reshape(N, 1024)
    h = _bn_relu(h, g0, b0)
    h = _gat_layer(h, src, dst, deg_o_isqrt, deg_i_sqrt, W1, al1, ar1, rW1,
                   4, 256).reshape(N, 1024)
    h = _bn_relu(h, g1, b1)
    h = _gat_layer(h, src, dst, deg_o_isqrt, deg_i_sqrt, W2, al2, ar2, rW2,
                   1, 40)
    h = jnp.mean(h, axis=1) + bias_last
    return h


# speedup vs baseline: 4.8911x; 4.8911x over previous
import functools

import jax
import jax.numpy as jnp
from jax import lax
from jax.experimental import pallas as pl
from jax.experimental.pallas import tpu as pltpu
from jax.experimental.pallas import tpu_sc as plsc

N = 10000
E = 160000
EPS = 1e-5

NW = 32
NPW = 320
NPAD = NW * NPW
RB = 512


def _mm_body(x_ref, w_ref, o_ref):
    o_ref[...] = jnp.dot(x_ref[...], w_ref[...],
                         preferred_element_type=jnp.float32)


def _mm(x, w):
    n, k = x.shape
    m = w.shape[1]
    return pl.pallas_call(
        _mm_body,
        grid=(n // RB,),
        in_specs=[
            pl.BlockSpec((RB, k), lambda i: (i, 0)),
            pl.BlockSpec((k, m), lambda i: (0, 0)),
        ],
        out_specs=pl.BlockSpec((RB, m), lambda i: (i, 0)),
        out_shape=jax.ShapeDtypeStruct((n, m), jnp.float32),
    )(x, w)


def _scale_body(x_ref, s_ref, o_ref):
    o_ref[...] = x_ref[...] * s_ref[...]


def _scale_rows(x, s):
    n, d = x.shape
    return pl.pallas_call(
        _scale_body,
        grid=(n // RB,),
        in_specs=[
            pl.BlockSpec((RB, d), lambda i: (i, 0)),
            pl.BlockSpec((RB, 1), lambda i: (i, 0)),
        ],
        out_specs=pl.BlockSpec((RB, d), lambda i: (i, 0)),
        out_shape=jax.ShapeDtypeStruct((n, d), jnp.float32),
    )(x, s)


def _stats_body(rst_ref, s_ref, res_ref, y_ref, sum_ref, sq_ref):
    i = pl.program_id(0)
    y = jnp.maximum(rst_ref[...] * s_ref[...] + res_ref[...], 0.0)
    y_ref[...] = y

    @pl.when(i == 0)
    def _():
        sum_ref[...] = jnp.zeros_like(sum_ref)
        sq_ref[...] = jnp.zeros_like(sq_ref)

    sum_ref[0:1, :] += jnp.sum(y, axis=0, keepdims=True)
    sq_ref[0:1, :] += jnp.sum(y * y, axis=0, keepdims=True)


def _post_stats(rst, s_i, res):
    n, d = rst.shape
    return pl.pallas_call(
        _stats_body,
        grid=(n // RB,),
        in_specs=[
            pl.BlockSpec((RB, d), lambda i: (i, 0)),
            pl.BlockSpec((RB, 1), lambda i: (i, 0)),
            pl.BlockSpec((RB, d), lambda i: (i, 0)),
        ],
        out_specs=[
            pl.BlockSpec((RB, d), lambda i: (i, 0)),
            pl.BlockSpec((8, d), lambda i: (0, 0)),
            pl.BlockSpec((8, d), lambda i: (0, 0)),
        ],
        out_shape=[
            jax.ShapeDtypeStruct((n, d), jnp.float32),
            jax.ShapeDtypeStruct((8, d), jnp.float32),
            jax.ShapeDtypeStruct((8, d), jnp.float32),
        ],
    )(rst, s_i, res)


def _apply_body(y_ref, sum_ref, sq_ref, g_ref, b_ref, so_ref, o_ref):
    m = sum_ref[0:1, :] * (1.0 / N)
    v = sq_ref[0:1, :] * (1.0 / N) - m * m
    z = (y_ref[...] - m) * lax.rsqrt(v + EPS) * g_ref[0:1, :] + b_ref[0:1, :]
    o_ref[...] = jnp.maximum(z, 0.0) * so_ref[...]


def _bn_apply(y, sums, sq, g8, b8, s_o):
    n, d = y.shape
    return pl.pallas_call(
        _apply_body,
        grid=(n // RB,),
        in_specs=[
            pl.BlockSpec((RB, d), lambda i: (i, 0)),
            pl.BlockSpec((8, d), lambda i: (0, 0)),
            pl.BlockSpec((8, d), lambda i: (0, 0)),
            pl.BlockSpec((8, d), lambda i: (0, 0)),
            pl.BlockSpec((8, d), lambda i: (0, 0)),
            pl.BlockSpec((RB, 1), lambda i: (i, 0)),
        ],
        out_specs=pl.BlockSpec((RB, d), lambda i: (i, 0)),
        out_shape=jax.ShapeDtypeStruct((n, d), jnp.float32),
    )(y, sums, sq, g8, b8, s_o)


def _final_body(rst_ref, s_ref, res_ref, bias_ref, o_ref):
    y = jnp.maximum(rst_ref[...] * s_ref[...] + res_ref[...], 0.0)
    o_ref[...] = y + bias_ref[0:1, :]


def _final(rst, s_i, res, bias8):
    n, d = rst.shape
    return pl.pallas_call(
        _final_body,
        grid=(n // RB,),
        in_specs=[
            pl.BlockSpec((RB, d), lambda i: (i, 0)),
            pl.BlockSpec((RB, 1), lambda i: (i, 0)),
            pl.BlockSpec((RB, d), lambda i: (i, 0)),
            pl.BlockSpec((8, d), lambda i: (0, 0)),
        ],
        out_specs=pl.BlockSpec((RB, d), lambda i: (i, 0)),
        out_shape=jax.ShapeDtypeStruct((n, d), jnp.float32),
    )(rst, s_i, res, bias8)


def _lane_bcast(vec, idx_vec):
    dn = lax.GatherDimensionNumbers(offset_dims=(), collapsed_slice_dims=(0,),
                                    start_index_map=(0,))
    return lax.gather(vec, idx_vec[:, None], dn, slice_sizes=(1,),
                      mode=lax.GatherScatterMode.PROMISE_IN_BOUNDS)


def _make_edge_agg(H, D):
    OUT = D // H
    NSL = D // 16
    mesh = plsc.VectorSubcoreMesh(core_axis_name="c", subcore_axis_name="s")

    def body(ft_hbm, el_hbm, er_hbm, rp_hbm, ssrc_hbm, out_hbm,
             el_v, er_v, rp_v, cbuf, rows_v, acc, sem):
        cidx = lax.axis_index("c")
        sidx = lax.axis_index("s")
        wid = sidx * 2 + cidx
        n0 = wid * NPW

        pltpu.sync_copy(el_hbm, el_v)
        pltpu.sync_copy(er_hbm.at[pl.ds(n0 * H, NPW * H + 16)], er_v)
        pltpu.sync_copy(rp_hbm.at[pl.ds(n0, NPW + 16)], rp_v)

        iota = lax.iota(jnp.int32, 16)

        def load_svec(pos):
            ab = (pos // 8) * 8
            off = pos - ab
            pltpu.sync_copy(ssrc_hbm.at[pl.ds(ab, 24)], cbuf)
            return cbuf[pl.ds(off, 16)]

        def evals(svec, er_b):
            es = []
            for h in range(H):
                g = plsc.load_gather(el_v, [svec * H + h])
                v = g + er_b[h]
                es.append(jnp.where(v >= 0.0, v, 0.2 * v))
            return es

        def node_body(ld, _):
            rpv = rp_v[pl.ds(ld, 16)]
            start = rpv[0]
            end = rpv[1]
            deg = end - start

            for sl in range(NSL):
                acc[pl.ds(sl * 16, 16)] = jnp.zeros((16,), jnp.float32)

            @pl.when(deg > 0)
            def _():
                ev = er_v[pl.ds(ld * H, 16)]
                er_b = [jnp.full((16,), ev[h]) for h in range(H)]
                nch = (deg + 15) // 16

                def pa(c, carry):
                    ms, dens = carry
                    pos = start + c * 16
                    rem = deg - c * 16
                    mask = iota < rem
                    svec = load_svec(pos)
                    es = evals(svec, er_b)
                    nm, nd = [], []
                    for h in range(H):
                        e = jnp.where(mask, es[h], -1e30)
                        m2 = jnp.maximum(ms[h], e)
                        d2 = dens[h] * jnp.exp(ms[h] - m2) + jnp.exp(e - m2)
                        nm.append(m2)
                        nd.append(d2)
                    return (tuple(nm), tuple(nd))

                init = (tuple(jnp.full((16,), -1e30) for _ in range(H)),
                        tuple(jnp.zeros((16,), jnp.float32) for _ in range(H)))
                ms, dens = lax.fori_loop(0, nch, pa, init)

                mst, invd = [], []
                for h in range(H):
                    m = jnp.max(ms[h])
                    denv = jnp.full((16,), jnp.sum(dens[h] * jnp.exp(ms[h] - m)))
                    mst.append(jnp.full((16,), m))
                    invd.append(1.0 / denv)

                def pb(c, _c):
                    pos = start + c * 16
                    rem = deg - c * 16
                    mask = iota < rem
                    svec = load_svec(pos)
                    pltpu.async_copy(ft_hbm.at[svec], rows_v, sem).wait()
                    es = evals(svec, er_b)
                    alphas = [
                        jnp.where(mask, jnp.exp(es[h] - mst[h]) * invd[h], 0.0)
                        for h in range(H)
                    ]

                    def eb(e2, _e):
                        e2v = jnp.full((16,), e2)
                        for h in range(H):
                            av = _lane_bcast(alphas[h], e2v)
                            for sl in range(OUT // 16):
                                o2 = h * OUT + sl * 16
                                acc[pl.ds(o2, 16)] += (
                                    av * rows_v[e2, pl.ds(o2, 16)])
                        return 0

                    lax.fori_loop(0, jnp.minimum(rem, 16), eb, 0)
                    return 0

                lax.fori_loop(0, nch, pb, 0)

            pltpu.sync_copy(acc, out_hbm.at[n0 + ld])
            return 0

        lax.fori_loop(0, NPW, node_body, 0)

    return pl.kernel(
        body,
        out_type=jax.ShapeDtypeStruct((NPAD, D), jnp.float32),
        mesh=mesh,
        compiler_params=pltpu.CompilerParams(needs_layout_passes=False),
        scratch_types=[
            pltpu.VMEM((N * H,), jnp.float32),
            pltpu.VMEM((NPW * H + 16,), jnp.float32),
            pltpu.VMEM((NPW + 16,), jnp.int32),
            pltpu.VMEM((24,), jnp.int32),
            pltpu.VMEM((16, D), jnp.float32),
            pltpu.VMEM((D,), jnp.float32),
            pltpu.SemaphoreType.DMA,
        ],
    )


_edge_agg_4 = _make_edge_agg(4, 1024)
_edge_agg_1 = _make_edge_agg(1, 128)


def _build_alar(al, ar, K, H, OUT):
    A = jnp.zeros((K, 256), jnp.float32)
    for h in range(H):
        A = A.at[h * OUT:(h + 1) * OUT, h].set(al[h])
        A = A.at[h * OUT:(h + 1) * OUT, 128 + h].set(ar[h])
    return A


def _pad8(v, d):
    return jnp.broadcast_to(v[None, :], (8, d))


def kernel(x, edge_index, W0, al0, ar0, rW0, g0, b0, W1, al1, ar1, rW1,
           g1, b1, W2, al2, ar2, rW2, bias_last):
    src = edge_index[0]
    dst = edge_index[1]

    key = (dst.astype(jnp.uint32) << 18) | jnp.arange(E, dtype=jnp.uint32)
    skey = jnp.sort(key)
    sdst = (skey >> 18).astype(jnp.int32)
    perm = (skey & 0x3FFFF).astype(jnp.int32)
    ssrc = src[perm]
    ssrc_pad = jnp.pad(ssrc, (0, 64))
    rp = jnp.searchsorted(sdst, jnp.arange(NPAD + 1, dtype=jnp.int32),
                          side="left").astype(jnp.int32)
    rp_pad = jnp.pad(rp, (0, 15), constant_values=E)

    deg_i = (rp[1:N + 1] - rp[:N]).astype(jnp.float32)
    s_i = jnp.sqrt(jnp.maximum(deg_i, 1.0))
    s_i = jnp.pad(s_i, (0, NPAD - N))[:, None]

    ssorted = jnp.sort(src)
    cnt = jnp.searchsorted(ssorted, jnp.arange(N + 1, dtype=jnp.int32),
                           side="left").astype(jnp.float32)
    deg_o = cnt[1:] - cnt[:-1]
    s_o = jnp.maximum(deg_o, 1.0) ** -0.5
    s_o = jnp.pad(s_o, (0, NPAD - N))[:, None]

    x_pad = jnp.pad(x, ((0, NPAD - N), (0, 0)))

    A0 = _build_alar(al0, ar0, 1024, 4, 256)
    A1 = _build_alar(al1, ar1, 1024, 4, 256)
    A2 = _build_alar(al2, ar2, 128, 1, 40)
    W2p = jnp.pad(W2, ((0, 0), (0, 88)))
    rW2p = jnp.pad(rW2, ((0, 0), (0, 88)))
    bias8 = _pad8(jnp.pad(bias_last, (0, 88)), 128)

    feat = _scale_rows(x_pad, s_o)
    ft = _mm(feat, W0)
    res = _mm(feat, rW0)
    elr = _mm(ft, A0)
    el_flat = elr[:N, 0:4].reshape(N * 4)
    er_flat = jnp.pad(elr[:, 128:132].reshape(NPAD * 4), (0, 16))
    rst = _edge_agg_4(ft, el_flat, er_flat, rp_pad, ssrc_pad)
    y, sums, sq = _post_stats(rst, s_i, res)
    feat = _bn_apply(y, sums, sq, _pad8(g0, 1024), _pad8(b0, 1024), s_o)

    ft = _mm(feat, W1)
    res = _mm(feat, rW1)
    elr = _mm(ft, A1)
    el_flat = elr[:N, 0:4].reshape(N * 4)
    er_flat = jnp.pad(elr[:, 128:132].reshape(NPAD * 4), (0, 16))
    rst = _edge_agg_4(ft, el_flat, er_flat, rp_pad, ssrc_pad)
    y, sums, sq = _post_stats(rst, s_i, res)
    feat = _bn_apply(y, sums, sq, _pad8(g1, 1024), _pad8(b1, 1024), s_o)

    ft = _mm(feat, W2p)
    res = _mm(feat, rW2p)
    elr = _mm(ft, A2)
    el_flat = elr[:N, 0:1].reshape(N)
    er_flat = jnp.pad(elr[:, 128:129].reshape(NPAD), (0, 16))
    rst = _edge_agg_1(ft, el_flat, er_flat, rp_pad, ssrc_pad)
    out = _final(rst, s_i, res, bias8)
    return out[:N, :40]


# speedup vs baseline: 9.0934x; 1.8592x over previous
import functools

import jax
import jax.numpy as jnp
from jax import lax
from jax.experimental import pallas as pl
from jax.experimental.pallas import tpu as pltpu
from jax.experimental.pallas import tpu_sc as plsc

N = 10000
E = 160000
EPS = 1e-5

NW = 32
NPW = 320
NPAD = NW * NPW
RB = 512


def _mm_body(x_ref, w_ref, o_ref):
    o_ref[...] = jnp.dot(x_ref[...], w_ref[...],
                         preferred_element_type=jnp.float32)


def _mm(x, w):
    n, k = x.shape
    m = w.shape[1]
    return pl.pallas_call(
        _mm_body,
        grid=(n // RB,),
        in_specs=[
            pl.BlockSpec((RB, k), lambda i: (i, 0)),
            pl.BlockSpec((k, m), lambda i: (0, 0)),
        ],
        out_specs=pl.BlockSpec((RB, m), lambda i: (i, 0)),
        out_shape=jax.ShapeDtypeStruct((n, m), jnp.float32),
    )(x, w)


def _scale_body(x_ref, s_ref, o_ref):
    o_ref[...] = x_ref[...] * s_ref[...]


def _scale_rows(x, s):
    n, d = x.shape
    return pl.pallas_call(
        _scale_body,
        grid=(n // RB,),
        in_specs=[
            pl.BlockSpec((RB, d), lambda i: (i, 0)),
            pl.BlockSpec((RB, 1), lambda i: (i, 0)),
        ],
        out_specs=pl.BlockSpec((RB, d), lambda i: (i, 0)),
        out_shape=jax.ShapeDtypeStruct((n, d), jnp.float32),
    )(x, s)


def _stats_body(rst_ref, s_ref, res_ref, y_ref, sum_ref, sq_ref):
    i = pl.program_id(0)
    y = jnp.maximum(rst_ref[...] * s_ref[...] + res_ref[...], 0.0)
    y_ref[...] = y

    @pl.when(i == 0)
    def _():
        sum_ref[...] = jnp.zeros_like(sum_ref)
        sq_ref[...] = jnp.zeros_like(sq_ref)

    sum_ref[0:1, :] += jnp.sum(y, axis=0, keepdims=True)
    sq_ref[0:1, :] += jnp.sum(y * y, axis=0, keepdims=True)


def _post_stats(rst, s_i, res):
    n, d = rst.shape
    return pl.pallas_call(
        _stats_body,
        grid=(n // RB,),
        in_specs=[
            pl.BlockSpec((RB, d), lambda i: (i, 0)),
            pl.BlockSpec((RB, 1), lambda i: (i, 0)),
            pl.BlockSpec((RB, d), lambda i: (i, 0)),
        ],
        out_specs=[
            pl.BlockSpec((RB, d), lambda i: (i, 0)),
            pl.BlockSpec((8, d), lambda i: (0, 0)),
            pl.BlockSpec((8, d), lambda i: (0, 0)),
        ],
        out_shape=[
            jax.ShapeDtypeStruct((n, d), jnp.float32),
            jax.ShapeDtypeStruct((8, d), jnp.float32),
            jax.ShapeDtypeStruct((8, d), jnp.float32),
        ],
    )(rst, s_i, res)


def _apply_body(y_ref, sum_ref, sq_ref, g_ref, b_ref, so_ref, o_ref):
    m = sum_ref[0:1, :] * (1.0 / N)
    v = sq_ref[0:1, :] * (1.0 / N) - m * m
    z = (y_ref[...] - m) * lax.rsqrt(v + EPS) * g_ref[0:1, :] + b_ref[0:1, :]
    o_ref[...] = jnp.maximum(z, 0.0) * so_ref[...]


def _bn_apply(y, sums, sq, g8, b8, s_o):
    n, d = y.shape
    return pl.pallas_call(
        _apply_body,
        grid=(n // RB,),
        in_specs=[
            pl.BlockSpec((RB, d), lambda i: (i, 0)),
            pl.BlockSpec((8, d), lambda i: (0, 0)),
            pl.BlockSpec((8, d), lambda i: (0, 0)),
            pl.BlockSpec((8, d), lambda i: (0, 0)),
            pl.BlockSpec((8, d), lambda i: (0, 0)),
            pl.BlockSpec((RB, 1), lambda i: (i, 0)),
        ],
        out_specs=pl.BlockSpec((RB, d), lambda i: (i, 0)),
        out_shape=jax.ShapeDtypeStruct((n, d), jnp.float32),
    )(y, sums, sq, g8, b8, s_o)


def _final_body(rst_ref, s_ref, res_ref, bias_ref, o_ref):
    y = jnp.maximum(rst_ref[...] * s_ref[...] + res_ref[...], 0.0)
    o_ref[...] = y + bias_ref[0:1, :]


def _final(rst, s_i, res, bias8):
    n, d = rst.shape
    return pl.pallas_call(
        _final_body,
        grid=(n // RB,),
        in_specs=[
            pl.BlockSpec((RB, d), lambda i: (i, 0)),
            pl.BlockSpec((RB, 1), lambda i: (i, 0)),
            pl.BlockSpec((RB, d), lambda i: (i, 0)),
            pl.BlockSpec((8, d), lambda i: (0, 0)),
        ],
        out_specs=pl.BlockSpec((RB, d), lambda i: (i, 0)),
        out_shape=jax.ShapeDtypeStruct((n, d), jnp.float32),
    )(rst, s_i, res, bias8)


def _lane_bcast(vec, idx_vec):
    dn = lax.GatherDimensionNumbers(offset_dims=(), collapsed_slice_dims=(0,),
                                    start_index_map=(0,))
    return lax.gather(vec, idx_vec[:, None], dn, slice_sizes=(1,),
                      mode=lax.GatherScatterMode.PROMISE_IN_BOUNDS)


CAP = 8192


def _make_edge_agg(H, D):
    OUT = D // H
    NSL = D // 16
    SLH = OUT // 16
    NG = NSL // 8
    mesh = plsc.VectorSubcoreMesh(core_axis_name="c", subcore_axis_name="s")

    def body(ft_hbm, el_hbm, er_hbm, rp_hbm, ssrc_hbm, out_hbm,
             el_v, er_v, rp_v, sbuf, cbuf, rows_v, acc8, sem):
        cidx = lax.axis_index("c")
        sidx = lax.axis_index("s")
        wid = sidx * 2 + cidx
        n0 = wid * NPW

        pltpu.sync_copy(el_hbm, el_v)
        pltpu.sync_copy(er_hbm.at[pl.ds(n0 * H, NPW * H + 16)], er_v)
        pltpu.sync_copy(rp_hbm.at[pl.ds(n0, NPW + 16)], rp_v)

        abase = (rp_v[pl.ds(0, 16)][0] // 8) * 8
        pltpu.sync_copy(ssrc_hbm.at[pl.ds(abase, CAP + 8)], sbuf)

        iota = lax.iota(jnp.int32, 16)

        def load_svec(pos):
            fast = (pos + 16) <= (abase + CAP + 8)

            @pl.when(jnp.logical_not(fast))
            def _():
                ab = (pos // 8) * 8
                pltpu.sync_copy(ssrc_hbm.at[pl.ds(ab, 24)], cbuf)

            v_fast = sbuf[pl.ds(jnp.where(fast, pos - abase, 0), 16)]
            v_slow = cbuf[pl.ds(jnp.where(fast, 0, pos - (pos // 8) * 8), 16)]
            return jnp.where(fast, v_fast, v_slow)

        def evals(svec, er_b):
            es = []
            for h in range(H):
                g = plsc.load_gather(el_v, [svec * H + h])
                v = g + er_b[h]
                es.append(jnp.where(v >= 0.0, v, 0.2 * v))
            return es

        def node_body(ld, _):
            rpv = rp_v[pl.ds(ld, 16)]
            start = rpv[0]
            end = rpv[1]
            deg = end - start
            nd8 = lax.rem(ld, 8)

            for sl in range(NSL):
                acc8[nd8, pl.ds(sl * 16, 16)] = jnp.zeros((16,), jnp.float32)

            @pl.when(deg > 0)
            def _():
                ev = er_v[pl.ds(ld * H, 16)]
                er_b = [jnp.full((16,), ev[h]) for h in range(H)]
                nch = (deg + 15) // 16

                def pa(c, carry):
                    ms, dens = carry
                    pos = start + c * 16
                    rem = deg - c * 16
                    mask = iota < rem
                    svec = load_svec(pos)
                    es = evals(svec, er_b)
                    nm, nd = [], []
                    for h in range(H):
                        e = jnp.where(mask, es[h], -1e30)
                        m2 = jnp.maximum(ms[h], e)
                        d2 = dens[h] * jnp.exp(ms[h] - m2) + jnp.exp(e - m2)
                        nm.append(m2)
                        nd.append(d2)
                    return (tuple(nm), tuple(nd))

                init = (tuple(jnp.full((16,), -1e30) for _ in range(H)),
                        tuple(jnp.zeros((16,), jnp.float32) for _ in range(H)))
                ms, dens = lax.fori_loop(0, nch, pa, init)

                mst, invd = [], []
                for h in range(H):
                    m = jnp.max(ms[h])
                    denv = jnp.full((16,), jnp.sum(dens[h] * jnp.exp(ms[h] - m)))
                    mst.append(jnp.full((16,), m))
                    invd.append(1.0 / denv)

                def pb(c, _c):
                    pos = start + c * 16
                    rem = deg - c * 16
                    mask = iota < rem
                    svec = load_svec(pos)
                    pltpu.async_copy(ft_hbm.at[svec], rows_v, sem).wait()
                    es = evals(svec, er_b)
                    alphas = [
                        jnp.where(mask, jnp.exp(es[h] - mst[h]) * invd[h], 0.0)
                        for h in range(H)
                    ]

                    for g in range(NG):
                        h = (g * 8) // SLH
                        base = g * 128

                        def eb(e2, accs):
                            av = _lane_bcast(alphas[h],
                                             jnp.full((16,), e2, jnp.int32))
                            return tuple(
                                accs[j] + av * rows_v[e2, pl.ds(base + j * 16,
                                                                16)]
                                for j in range(8))

                        accs = tuple(acc8[nd8, pl.ds(base + j * 16, 16)]
                                     for j in range(8))
                        accs = lax.fori_loop(0, 16, eb, accs)
                        for j in range(8):
                            acc8[nd8, pl.ds(base + j * 16, 16)] = accs[j]
                    return 0

                lax.fori_loop(0, nch, pb, 0)

            @pl.when(nd8 == 7)
            def _():
                row0 = pl.multiple_of(n0 + ld - 7, 8)
                pltpu.sync_copy(acc8, out_hbm.at[pl.ds(row0, 8)])
            return 0

        lax.fori_loop(0, NPW, node_body, 0)

    return pl.kernel(
        body,
        out_type=jax.ShapeDtypeStruct((NPAD, D), jnp.float32),
        mesh=mesh,
        compiler_params=pltpu.CompilerParams(needs_layout_passes=False),
        scratch_types=[
            pltpu.VMEM((N * H,), jnp.float32),
            pltpu.VMEM((NPW * H + 16,), jnp.float32),
            pltpu.VMEM((NPW + 16,), jnp.int32),
            pltpu.VMEM((CAP + 8,), jnp.int32),
            pltpu.VMEM((24,), jnp.int32),
            pltpu.VMEM((16, D), jnp.float32),
            pltpu.VMEM((8, D), jnp.float32),
            pltpu.SemaphoreType.DMA,
        ],
    )


_edge_agg_4 = _make_edge_agg(4, 1024)
_edge_agg_1 = _make_edge_agg(1, 128)


def _build_alar(al, ar, K, H, OUT):
    A = jnp.zeros((K, 256), jnp.float32)
    for h in range(H):
        A = A.at[h * OUT:(h + 1) * OUT, h].set(al[h])
        A = A.at[h * OUT:(h + 1) * OUT, 128 + h].set(ar[h])
    return A


def _pad8(v, d):
    return jnp.broadcast_to(v[None, :], (8, d))


def kernel(x, edge_index, W0, al0, ar0, rW0, g0, b0, W1, al1, ar1, rW1,
           g1, b1, W2, al2, ar2, rW2, bias_last):
    src = edge_index[0]
    dst = edge_index[1]

    key = (dst.astype(jnp.uint32) << 18) | jnp.arange(E, dtype=jnp.uint32)
    skey = jnp.sort(key)
    sdst = (skey >> 18).astype(jnp.int32)
    perm = (skey & 0x3FFFF).astype(jnp.int32)
    ssrc = src[perm]
    ssrc_pad = jnp.pad(ssrc, (0, CAP + 24))
    rp = jnp.searchsorted(sdst, jnp.arange(NPAD + 1, dtype=jnp.int32),
                          side="left").astype(jnp.int32)
    rp_pad = jnp.pad(rp, (0, 15), constant_values=E)

    deg_i = (rp[1:N + 1] - rp[:N]).astype(jnp.float32)
    s_i = jnp.sqrt(jnp.maximum(deg_i, 1.0))
    s_i = jnp.pad(s_i, (0, NPAD - N))[:, None]

    ssorted = jnp.sort(src)
    cnt = jnp.searchsorted(ssorted, jnp.arange(N + 1, dtype=jnp.int32),
                           side="left").astype(jnp.float32)
    deg_o = cnt[1:] - cnt[:-1]
    s_o = jnp.maximum(deg_o, 1.0) ** -0.5
    s_o = jnp.pad(s_o, (0, NPAD - N))[:, None]

    x_pad = jnp.pad(x, ((0, NPAD - N), (0, 0)))

    A0 = _build_alar(al0, ar0, 1024, 4, 256)
    A1 = _build_alar(al1, ar1, 1024, 4, 256)
    A2 = _build_alar(al2, ar2, 128, 1, 40)
    W2p = jnp.pad(W2, ((0, 0), (0, 88)))
    rW2p = jnp.pad(rW2, ((0, 0), (0, 88)))
    bias8 = _pad8(jnp.pad(bias_last, (0, 88)), 128)

    feat = _scale_rows(x_pad, s_o)
    ft = _mm(feat, W0)
    res = _mm(feat, rW0)
    elr = _mm(ft, A0)
    el_flat = elr[:N, 0:4].reshape(N * 4)
    er_flat = jnp.pad(elr[:, 128:132].reshape(NPAD * 4), (0, 16))
    rst = _edge_agg_4(ft, el_flat, er_flat, rp_pad, ssrc_pad)
    y, sums, sq = _post_stats(rst, s_i, res)
    feat = _bn_apply(y, sums, sq, _pad8(g0, 1024), _pad8(b0, 1024), s_o)

    ft = _mm(feat, W1)
    res = _mm(feat, rW1)
    elr = _mm(ft, A1)
    el_flat = elr[:N, 0:4].reshape(N * 4)
    er_flat = jnp.pad(elr[:, 128:132].reshape(NPAD * 4), (0, 16))
    rst = _edge_agg_4(ft, el_flat, er_flat, rp_pad, ssrc_pad)
    y, sums, sq = _post_stats(rst, s_i, res)
    feat = _bn_apply(y, sums, sq, _pad8(g1, 1024), _pad8(b1, 1024), s_o)

    ft = _mm(feat, W2p)
    res = _mm(feat, rW2p)
    elr = _mm(ft, A2)
    el_flat = elr[:N, 0:1].reshape(N)
    er_flat = jnp.pad(elr[:, 128:129].reshape(NPAD), (0, 16))
    rst = _edge_agg_1(ft, el_flat, er_flat, rp_pad, ssrc_pad)
    out = _final(rst, s_i, res, bias8)
    return out[:N, :40]


# speedup vs baseline: 10.6250x; 1.1684x over previous
import functools

import jax
import jax.numpy as jnp
from jax import lax
from jax.experimental import pallas as pl
from jax.experimental.pallas import tpu as pltpu
from jax.experimental.pallas import tpu_sc as plsc

N = 10000
E = 160000
EPS = 1e-5

NW = 32
NPW = 320
NPAD = NW * NPW
RB = 512


def _mm_body(x_ref, w_ref, o_ref):
    o_ref[...] = jnp.dot(x_ref[...], w_ref[...],
                         preferred_element_type=jnp.float32)


def _mm(x, w):
    n, k = x.shape
    m = w.shape[1]
    return pl.pallas_call(
        _mm_body,
        grid=(n // RB,),
        in_specs=[
            pl.BlockSpec((RB, k), lambda i: (i, 0)),
            pl.BlockSpec((k, m), lambda i: (0, 0)),
        ],
        out_specs=pl.BlockSpec((RB, m), lambda i: (i, 0)),
        out_shape=jax.ShapeDtypeStruct((n, m), jnp.float32),
    )(x, w)


def _scale_body(x_ref, s_ref, o_ref):
    o_ref[...] = x_ref[...] * s_ref[...]


def _scale_rows(x, s):
    n, d = x.shape
    return pl.pallas_call(
        _scale_body,
        grid=(n // RB,),
        in_specs=[
            pl.BlockSpec((RB, d), lambda i: (i, 0)),
            pl.BlockSpec((RB, 1), lambda i: (i, 0)),
        ],
        out_specs=pl.BlockSpec((RB, d), lambda i: (i, 0)),
        out_shape=jax.ShapeDtypeStruct((n, d), jnp.float32),
    )(x, s)


def _stats_body(rst_ref, s_ref, res_ref, y_ref, sum_ref, sq_ref):
    i = pl.program_id(0)
    y = jnp.maximum(rst_ref[...] * s_ref[...] + res_ref[...], 0.0)
    y_ref[...] = y

    @pl.when(i == 0)
    def _():
        sum_ref[...] = jnp.zeros_like(sum_ref)
        sq_ref[...] = jnp.zeros_like(sq_ref)

    sum_ref[0:1, :] += jnp.sum(y, axis=0, keepdims=True)
    sq_ref[0:1, :] += jnp.sum(y * y, axis=0, keepdims=True)


def _post_stats(rst, s_i, res):
    n, d = rst.shape
    return pl.pallas_call(
        _stats_body,
        grid=(n // RB,),
        in_specs=[
            pl.BlockSpec((RB, d), lambda i: (i, 0)),
            pl.BlockSpec((RB, 1), lambda i: (i, 0)),
            pl.BlockSpec((RB, d), lambda i: (i, 0)),
        ],
        out_specs=[
            pl.BlockSpec((RB, d), lambda i: (i, 0)),
            pl.BlockSpec((8, d), lambda i: (0, 0)),
            pl.BlockSpec((8, d), lambda i: (0, 0)),
        ],
        out_shape=[
            jax.ShapeDtypeStruct((n, d), jnp.float32),
            jax.ShapeDtypeStruct((8, d), jnp.float32),
            jax.ShapeDtypeStruct((8, d), jnp.float32),
        ],
    )(rst, s_i, res)


def _apply_body(y_ref, sum_ref, sq_ref, g_ref, b_ref, so_ref, o_ref):
    m = sum_ref[0:1, :] * (1.0 / N)
    v = sq_ref[0:1, :] * (1.0 / N) - m * m
    z = (y_ref[...] - m) * lax.rsqrt(v + EPS) * g_ref[0:1, :] + b_ref[0:1, :]
    o_ref[...] = jnp.maximum(z, 0.0) * so_ref[...]


def _bn_apply(y, sums, sq, g8, b8, s_o):
    n, d = y.shape
    return pl.pallas_call(
        _apply_body,
        grid=(n // RB,),
        in_specs=[
            pl.BlockSpec((RB, d), lambda i: (i, 0)),
            pl.BlockSpec((8, d), lambda i: (0, 0)),
            pl.BlockSpec((8, d), lambda i: (0, 0)),
            pl.BlockSpec((8, d), lambda i: (0, 0)),
            pl.BlockSpec((8, d), lambda i: (0, 0)),
            pl.BlockSpec((RB, 1), lambda i: (i, 0)),
        ],
        out_specs=pl.BlockSpec((RB, d), lambda i: (i, 0)),
        out_shape=jax.ShapeDtypeStruct((n, d), jnp.float32),
    )(y, sums, sq, g8, b8, s_o)


def _final_body(rst_ref, s_ref, res_ref, bias_ref, o_ref):
    y = jnp.maximum(rst_ref[...] * s_ref[...] + res_ref[...], 0.0)
    o_ref[...] = y + bias_ref[0:1, :]


def _final(rst, s_i, res, bias8):
    n, d = rst.shape
    return pl.pallas_call(
        _final_body,
        grid=(n // RB,),
        in_specs=[
            pl.BlockSpec((RB, d), lambda i: (i, 0)),
            pl.BlockSpec((RB, 1), lambda i: (i, 0)),
            pl.BlockSpec((RB, d), lambda i: (i, 0)),
            pl.BlockSpec((8, d), lambda i: (0, 0)),
        ],
        out_specs=pl.BlockSpec((RB, d), lambda i: (i, 0)),
        out_shape=jax.ShapeDtypeStruct((n, d), jnp.float32),
    )(rst, s_i, res, bias8)


def _lane_bcast(vec, idx_vec):
    dn = lax.GatherDimensionNumbers(offset_dims=(), collapsed_slice_dims=(0,),
                                    start_index_map=(0,))
    return lax.gather(vec, idx_vec[:, None], dn, slice_sizes=(1,),
                      mode=lax.GatherScatterMode.PROMISE_IN_BOUNDS)


CAP = 8192


def _make_edge_agg(H, D):
    OUT = D // H
    NSL = D // 16
    SLH = OUT // 16
    NG = NSL // 8
    mesh = plsc.VectorSubcoreMesh(core_axis_name="c", subcore_axis_name="s")

    def body(ft_hbm, el_hbm, er_hbm, rp_hbm, ssrc_hbm, out_hbm,
             el_v, er_v, rp_v, sbuf, cbuf, rows_v, acc8, sem0, sem1):
        cidx = lax.axis_index("c")
        sidx = lax.axis_index("s")
        wid = sidx * 2 + cidx
        n0 = wid * NPW

        pltpu.sync_copy(el_hbm, el_v)
        pltpu.sync_copy(er_hbm.at[pl.ds(n0 * H, NPW * H + 16)], er_v)
        pltpu.sync_copy(rp_hbm.at[pl.ds(n0, NPW + 16)], rp_v)

        abase = (rp_v[pl.ds(0, 16)][0] // 8) * 8
        pltpu.sync_copy(ssrc_hbm.at[pl.ds(abase, CAP + 8)], sbuf)

        iota = lax.iota(jnp.int32, 16)

        def load_svec(pos):
            fast = (pos + 16) <= (abase + CAP + 8)

            @pl.when(jnp.logical_not(fast))
            def _():
                ab = (pos // 8) * 8
                pltpu.sync_copy(ssrc_hbm.at[pl.ds(ab, 24)], cbuf)

            v_fast = sbuf[pl.ds(jnp.where(fast, pos - abase, 0), 16)]
            v_slow = cbuf[pl.ds(jnp.where(fast, 0, pos - (pos // 8) * 8), 16)]
            return jnp.where(fast, v_fast, v_slow)

        def issue(svec, par):
            @pl.when(par == 0)
            def _():
                pltpu.async_copy(ft_hbm.at[svec], rows_v.at[0], sem0)

            @pl.when(par == 1)
            def _():
                pltpu.async_copy(ft_hbm.at[svec], rows_v.at[1], sem1)

        def wait(svec, par):
            @pl.when(par == 0)
            def _():
                pltpu.make_async_copy(ft_hbm.at[svec], rows_v.at[0],
                                      sem0).wait()

            @pl.when(par == 1)
            def _():
                pltpu.make_async_copy(ft_hbm.at[svec], rows_v.at[1],
                                      sem1).wait()

        def evals(svec, er_b):
            es = []
            for h in range(H):
                g = plsc.load_gather(el_v, [svec * H + h])
                v = g + er_b[h]
                es.append(jnp.where(v >= 0.0, v, 0.2 * v))
            return es

        def node_body(ld, carry):
            qi, qc, startP, degP, er_bP, mstP, invdP = carry

            rpv = rp_v[pl.ds(ld, 16)]
            start = rpv[0]
            deg = rpv[1] - rpv[0]
            ev = er_v[pl.ds(ld * H, 16)]
            er_b = tuple(jnp.full((16,), ev[h]) for h in range(H))
            nch = (deg + 15) // 16

            def pa(c, acarry):
                ms, dens = acarry
                pos = start + c * 16
                rem = deg - c * 16
                mask = iota < rem
                svec = load_svec(pos)
                es = evals(svec, er_b)
                nm, nd = [], []
                for h in range(H):
                    e = jnp.where(mask, es[h], -1e30)
                    m2 = jnp.maximum(ms[h], e)
                    d2 = dens[h] * jnp.exp(ms[h] - m2) + jnp.exp(e - m2)
                    nm.append(m2)
                    nd.append(d2)
                return (tuple(nm), tuple(nd))

            init = (tuple(jnp.full((16,), -1e30) for _ in range(H)),
                    tuple(jnp.zeros((16,), jnp.float32) for _ in range(H)))
            ms, dens = lax.fori_loop(0, nch, pa, init)

            mst, invd = [], []
            for h in range(H):
                m = jnp.max(ms[h])
                denv = jnp.full((16,), jnp.sum(dens[h] * jnp.exp(ms[h] - m)))
                mst.append(jnp.full((16,), m))
                invd.append(1.0 / denv)
            mst = tuple(mst)
            invd = tuple(invd)

            nchP = (degP + 15) // 16

            pre = jnp.logical_and(jnp.logical_and(deg > 0, nchP <= 1),
                                  ld < NPW)

            @pl.when(pre)
            def _():
                issue(load_svec(start), qi)

            qi = jnp.where(pre, lax.rem(qi + 1, 2), qi)

            ldP = ld - 1
            nd8P = lax.rem(ldP + 8, 8)
            for sl in range(NSL):
                acc8[nd8P, pl.ds(sl * 16, 16)] = jnp.zeros((16,), jnp.float32)

            def pbchunk(c, qq):
                qi_c, qc_c = qq
                pos = startP + c * 16
                rem = degP - c * 16
                mask = iota < rem
                svec = load_svec(pos)
                nxt = c + 1

                @pl.when(nxt < nchP)
                def _():
                    issue(load_svec(startP + nxt * 16), qi_c)

                qi_n = jnp.where(nxt < nchP, lax.rem(qi_c + 1, 2), qi_c)
                wait(svec, qc_c)
                es = evals(svec, er_bP)
                alphas = [
                    jnp.where(mask, jnp.exp(es[h] - mstP[h]) * invdP[h], 0.0)
                    for h in range(H)
                ]

                for g in range(NG):
                    h = (g * 8) // SLH
                    base = g * 128

                    def eb(e2, accs):
                        av = _lane_bcast(alphas[h],
                                         jnp.full((16,), e2, jnp.int32))
                        return tuple(
                            accs[j] + av * rows_v[qc_c, e2,
                                                  pl.ds(base + j * 16, 16)]
                            for j in range(8))

                    accs = tuple(acc8[nd8P, pl.ds(base + j * 16, 16)]
                                 for j in range(8))
                    accs = lax.fori_loop(0, 16, eb, accs)
                    for j in range(8):
                        acc8[nd8P, pl.ds(base + j * 16, 16)] = accs[j]
                return (qi_n, lax.rem(qc_c + 1, 2))

            qi, qc = lax.fori_loop(0, nchP, pbchunk, (qi, qc))

            post = jnp.logical_and(jnp.logical_and(deg > 0, nchP > 1),
                                   ld < NPW)

            @pl.when(post)
            def _():
                issue(load_svec(start), qi)

            qi = jnp.where(post, lax.rem(qi + 1, 2), qi)

            @pl.when(jnp.logical_and(nd8P == 7, ld >= 1))
            def _():
                row0 = pl.multiple_of(n0 + ldP - 7, 8)
                pltpu.sync_copy(acc8, out_hbm.at[pl.ds(row0, 8)])

            return (qi, qc, start, deg, er_b, mst, invd)

        zero16 = jnp.zeros((16,), jnp.float32)
        init_carry = (jnp.int32(0), jnp.int32(0), jnp.int32(0), jnp.int32(0),
                      tuple(zero16 for _ in range(H)),
                      tuple(zero16 for _ in range(H)),
                      tuple(zero16 for _ in range(H)))
        lax.fori_loop(0, NPW + 1, node_body, init_carry)

    return pl.kernel(
        body,
        out_type=jax.ShapeDtypeStruct((NPAD, D), jnp.float32),
        mesh=mesh,
        compiler_params=pltpu.CompilerParams(needs_layout_passes=False),
        scratch_types=[
            pltpu.VMEM((N * H,), jnp.float32),
            pltpu.VMEM((NPW * H + 16,), jnp.float32),
            pltpu.VMEM((NPW + 16,), jnp.int32),
            pltpu.VMEM((CAP + 8,), jnp.int32),
            pltpu.VMEM((24,), jnp.int32),
            pltpu.VMEM((2, 16, D), jnp.float32),
            pltpu.VMEM((8, D), jnp.float32),
            pltpu.SemaphoreType.DMA,
            pltpu.SemaphoreType.DMA,
        ],
    )


_edge_agg_4 = _make_edge_agg(4, 1024)
_edge_agg_1 = _make_edge_agg(1, 128)


def _build_alar(al, ar, K, H, OUT):
    A = jnp.zeros((K, 256), jnp.float32)
    for h in range(H):
        A = A.at[h * OUT:(h + 1) * OUT, h].set(al[h])
        A = A.at[h * OUT:(h + 1) * OUT, 128 + h].set(ar[h])
    return A


def _pad8(v, d):
    return jnp.broadcast_to(v[None, :], (8, d))


def kernel(x, edge_index, W0, al0, ar0, rW0, g0, b0, W1, al1, ar1, rW1,
           g1, b1, W2, al2, ar2, rW2, bias_last):
    src = edge_index[0]
    dst = edge_index[1]

    key = (dst.astype(jnp.uint32) << 18) | jnp.arange(E, dtype=jnp.uint32)
    skey = jnp.sort(key)
    sdst = (skey >> 18).astype(jnp.int32)
    perm = (skey & 0x3FFFF).astype(jnp.int32)
    ssrc = src[perm]
    ssrc_pad = jnp.pad(ssrc, (0, CAP + 24))
    rp = jnp.searchsorted(sdst, jnp.arange(NPAD + 1, dtype=jnp.int32),
                          side="left").astype(jnp.int32)
    rp_pad = jnp.pad(rp, (0, 15), constant_values=E)

    deg_i = (rp[1:N + 1] - rp[:N]).astype(jnp.float32)
    s_i = jnp.sqrt(jnp.maximum(deg_i, 1.0))
    s_i = jnp.pad(s_i, (0, NPAD - N))[:, None]

    ssorted = jnp.sort(src)
    cnt = jnp.searchsorted(ssorted, jnp.arange(N + 1, dtype=jnp.int32),
                           side="left").astype(jnp.float32)
    deg_o = cnt[1:] - cnt[:-1]
    s_o = jnp.maximum(deg_o, 1.0) ** -0.5
    s_o = jnp.pad(s_o, (0, NPAD - N))[:, None]

    x_pad = jnp.pad(x, ((0, NPAD - N), (0, 0)))

    A0 = _build_alar(al0, ar0, 1024, 4, 256)
    A1 = _build_alar(al1, ar1, 1024, 4, 256)
    A2 = _build_alar(al2, ar2, 128, 1, 40)
    W2p = jnp.pad(W2, ((0, 0), (0, 88)))
    rW2p = jnp.pad(rW2, ((0, 0), (0, 88)))
    bias8 = _pad8(jnp.pad(bias_last, (0, 88)), 128)

    feat = _scale_rows(x_pad, s_o)
    ft = _mm(feat, W0)
    res = _mm(feat, rW0)
    elr = _mm(ft, A0)
    el_flat = elr[:N, 0:4].reshape(N * 4)
    er_flat = jnp.pad(elr[:, 128:132].reshape(NPAD * 4), (0, 16))
    rst = _edge_agg_4(ft, el_flat, er_flat, rp_pad, ssrc_pad)
    y, sums, sq = _post_stats(rst, s_i, res)
    feat = _bn_apply(y, sums, sq, _pad8(g0, 1024), _pad8(b0, 1024), s_o)

    ft = _mm(feat, W1)
    res = _mm(feat, rW1)
    elr = _mm(ft, A1)
    el_flat = elr[:N, 0:4].reshape(N * 4)
    er_flat = jnp.pad(elr[:, 128:132].reshape(NPAD * 4), (0, 16))
    rst = _edge_agg_4(ft, el_flat, er_flat, rp_pad, ssrc_pad)
    y, sums, sq = _post_stats(rst, s_i, res)
    feat = _bn_apply(y, sums, sq, _pad8(g1, 1024), _pad8(b1, 1024), s_o)

    ft = _mm(feat, W2p)
    res = _mm(feat, rW2p)
    elr = _mm(ft, A2)
    el_flat = elr[:N, 0:1].reshape(N)
    er_flat = jnp.pad(elr[:, 128:129].reshape(NPAD), (0, 16))
    rst = _edge_agg_1(ft, el_flat, er_flat, rp_pad, ssrc_pad)
    out = _final(rst, s_i, res, bias8)
    return out[:N, :40]
